# Initial kernel scaffold; baseline (speedup 1.0000x reference)
import jax
import jax.numpy as jnp
from jax.experimental import pallas as pl


def kernel(x, edge_index, batch, W1_rel, b1_rel, W1_root, W2_rel, b2_rel, W2_root, W3_rel, b3_rel, W3_root, W4_rel, b4_rel, W4_root, fc_W, fc_b):
    raise NotImplementedError("write your pallas kernel here")


# speedup vs baseline: 6.4584x; 6.4584x over previous
import functools

import jax
import jax.numpy as jnp
from jax import lax
from jax.experimental import pallas as pl
from jax.experimental.pallas import tpu as pltpu
from jax.experimental.pallas import tpu_sc as plsc

_N = 10000
_G = 64
_NC = 2
_NS = 16
_NW = _NC * _NS
_CHUNK = 128
_NCH = 80
_EPAD = _NW * _NCH * _CHUNK
_ACC = 10112


_NPH = 5
_PCH = _NCH // _NPH


def _sc_edge_aggregate(y, src_t, dst_t, zeros):
  dout = y.shape[1]
  mesh = plsc.VectorSubcoreMesh(core_axis_name="c", subcore_axis_name="s")

  @functools.partial(
      pl.kernel,
      out_type=jax.ShapeDtypeStruct((_NC * _ACC, dout), jnp.float32),
      mesh=mesh,
      scratch_types=[
          pltpu.VMEM((_PCH, _CHUNK), jnp.int32),
          pltpu.VMEM((_PCH, _CHUNK), jnp.int32),
          pltpu.VMEM((_PCH, _CHUNK), jnp.int32),
          pltpu.VMEM((_PCH, _CHUNK), jnp.int32),
          pltpu.VMEM((_CHUNK, dout), jnp.float32),
          pltpu.VMEM((_CHUNK, dout), jnp.float32),
          pltpu.VMEM_SHARED((_ACC, dout), jnp.float32),
          pltpu.SemaphoreType.DMA,
          pltpu.SemaphoreType.DMA,
          pltpu.SemaphoreType.DMA,
          pltpu.SemaphoreType.DMA,
      ],
      compiler_params=pltpu.CompilerParams(use_tc_tiling_on_sc=False),
  )
  def k(y_hbm, src_hbm, dst_hbm, z_hbm, out_hbm, sb0, db0, sb1, db1, g0, g1,
        acc, isem0, isem1, gsem0, gsem1):
    c = lax.axis_index("c")
    s = lax.axis_index("s")
    wid = s * _NC + c
    pltpu.async_copy(src_hbm.at[wid, 0], sb0, isem0)
    pltpu.async_copy(dst_hbm.at[wid, 0], db0, isem0)
    pltpu.async_copy(src_hbm.at[wid, 1], sb1, isem1)
    pltpu.async_copy(dst_hbm.at[wid, 1], db1, isem1)
    zr = _ACC // _NS
    pltpu.sync_copy(z_hbm.at[pl.ds(s * zr, zr)], acc.at[pl.ds(s * zr, zr)])
    plsc.subcore_barrier()

    for p in range(_NPH):
      sb, db, isem = (sb0, db0, isem0) if p % 2 == 0 else (sb1, db1, isem1)
      pltpu.make_async_copy(src_hbm.at[wid, p], sb, isem).wait()
      pltpu.make_async_copy(dst_hbm.at[wid, p], db, isem).wait()
      pltpu.async_copy(y_hbm.at[sb.at[0]], g0, gsem0)
      pltpu.async_copy(y_hbm.at[sb.at[1]], g1, gsem1)

      def step(i, carry, sb=sb, db=db):
        for b, (g, gs) in enumerate(((g0, gsem0), (g1, gsem1))):
          j = i * 2 + b
          pltpu.make_async_copy(y_hbm.at[sb.at[j]], g, gs).wait()
          pltpu.sync_copy(g, acc.at[db.at[j]], add=True)

          @pl.when(j + 2 < _PCH)
          def _():
            pltpu.async_copy(y_hbm.at[sb.at[j + 2]], g, gs)
        return carry

      lax.fori_loop(0, _PCH // 2, step, 0)
      if p + 2 < _NPH:
        pltpu.async_copy(src_hbm.at[wid, p + 2], sb, isem)
        pltpu.async_copy(dst_hbm.at[wid, p + 2], db, isem)

    plsc.subcore_barrier()
    orow = _ACC // _NS
    pltpu.sync_copy(acc.at[pl.ds(s * orow, orow)],
                    out_hbm.at[pl.ds(c * _ACC + s * orow, orow)])

  return k(y, src_t, dst_t, zeros)


def _tc_first(x, w_rel, w_root):
  dout = w_rel.shape[1]

  def body(x_r, wr_r, wo_r, y_r, r_r):
    xv = x_r[...]
    y_r[...] = jnp.dot(xv, wr_r[...], preferred_element_type=jnp.float32)
    r_r[...] = jnp.dot(xv, wo_r[...], preferred_element_type=jnp.float32)

  return pl.pallas_call(
      body,
      out_shape=(jax.ShapeDtypeStruct((_N, dout), jnp.float32),
                 jax.ShapeDtypeStruct((_N, dout), jnp.float32)),
  )(x, w_rel, w_root)


def _tc_mid(P, r, b2d, w_rel, w_root):
  dout = w_rel.shape[1]

  def body(p_r, r_r, b_r, wr_r, wo_r, y_r, q_r):
    h = p_r[:_N, :] + p_r[_ACC:_ACC + _N, :] + b_r[...] + r_r[...]
    h = jnp.maximum(h, 0.0)
    y_r[...] = jnp.dot(h, wr_r[...], preferred_element_type=jnp.float32)
    q_r[...] = jnp.dot(h, wo_r[...], preferred_element_type=jnp.float32)

  return pl.pallas_call(
      body,
      out_shape=(jax.ShapeDtypeStruct((_N, dout), jnp.float32),
                 jax.ShapeDtypeStruct((_N, dout), jnp.float32)),
  )(P, r, b2d, w_rel, w_root)


def _tc_head(P, r, b2d, batch2d, fc_w, fc_b2d):
  def body(p_r, r_r, b_r, bat_r, w_r, c_r, o_r):
    h = p_r[:_N, :] + p_r[_ACC:_ACC + _N, :] + b_r[...] + r_r[...]
    gid = lax.broadcasted_iota(jnp.int32, (_G, 1), 0)
    onehot = (bat_r[...] == gid).astype(jnp.float32)
    sums = jnp.dot(onehot, h, preferred_element_type=jnp.float32)
    cnt = jnp.sum(onehot, axis=1, keepdims=True)
    pooled = sums / jnp.maximum(cnt, 1.0)
    o_r[...] = jnp.dot(pooled, w_r[...],
                       preferred_element_type=jnp.float32) + c_r[...]

  return pl.pallas_call(
      body,
      out_shape=jax.ShapeDtypeStruct((_G, 1), jnp.float32),
  )(P, r, b2d, batch2d, fc_w, fc_b2d)


def kernel(x, edge_index, batch, W1_rel, b1_rel, W1_root, W2_rel, b2_rel,
           W2_root, W3_rel, b3_rel, W3_root, W4_rel, b4_rel, W4_root,
           fc_W, fc_b):
  src = edge_index[0]
  dst = edge_index[1]
  pad = _EPAD - src.shape[0]
  src_t = jnp.concatenate(
      [src, jnp.zeros((pad,), jnp.int32)]).reshape(_NW, _NPH, _PCH, _CHUNK)
  dst_t = jnp.concatenate(
      [dst, jnp.full((pad,), _N, jnp.int32)]).reshape(_NW, _NPH, _PCH, _CHUNK)
  batch2d = batch.reshape(1, _N)

  y, r = _tc_first(x, W1_rel, W1_root)
  P = _sc_edge_aggregate(y, src_t, dst_t,
                         jnp.zeros((_ACC, W1_rel.shape[1]), jnp.float32))
  y, r = _tc_mid(P, r, b1_rel.reshape(1, -1), W2_rel, W2_root)
  P = _sc_edge_aggregate(y, src_t, dst_t,
                         jnp.zeros((_ACC, W2_rel.shape[1]), jnp.float32))
  y, r = _tc_mid(P, r, b2_rel.reshape(1, -1), W3_rel, W3_root)
  P = _sc_edge_aggregate(y, src_t, dst_t,
                         jnp.zeros((_ACC, W3_rel.shape[1]), jnp.float32))
  y, r = _tc_mid(P, r, b3_rel.reshape(1, -1), W4_rel, W4_root)
  P = _sc_edge_aggregate(y, src_t, dst_t,
                         jnp.zeros((_ACC, W4_rel.shape[1]), jnp.float32))
  return _tc_head(P, r, b4_rel.reshape(1, -1), batch2d, fc_W,
                  fc_b.reshape(1, 1))


# speedup vs baseline: 15.9440x; 2.4687x over previous
import functools

import jax
import jax.numpy as jnp
from jax import lax
from jax.experimental import pallas as pl
from jax.experimental.pallas import tpu as pltpu
from jax.experimental.pallas import tpu_sc as plsc

_N = 10000
_G = 64
_NC = 2
_NS = 16
_NW = _NC * _NS
_CHUNK = 128
_NCH = 80
_EPAD = _NW * _NCH * _CHUNK
_ACC = 10112


_NPH = 5
_PCH = _NCH // _NPH


def _sc_edge_aggregate(y, src_t, dst_t, zeros):
  dout = y.shape[1]
  mesh = plsc.VectorSubcoreMesh(core_axis_name="c", subcore_axis_name="s")

  @functools.partial(
      pl.kernel,
      out_type=jax.ShapeDtypeStruct((_NC * _ACC, dout), jnp.float32),
      mesh=mesh,
      scratch_types=[
          pltpu.VMEM((_PCH, _CHUNK), jnp.int32),
          pltpu.VMEM((_PCH, _CHUNK), jnp.int32),
          pltpu.VMEM((_PCH, _CHUNK), jnp.int32),
          pltpu.VMEM((_PCH, _CHUNK), jnp.int32),
          pltpu.VMEM((_CHUNK, dout), jnp.float32),
          pltpu.VMEM((_CHUNK, dout), jnp.float32),
          pltpu.VMEM_SHARED((_ACC, dout), jnp.float32),
          pltpu.SemaphoreType.DMA,
          pltpu.SemaphoreType.DMA,
          pltpu.SemaphoreType.DMA,
          pltpu.SemaphoreType.DMA,
      ],
      compiler_params=pltpu.CompilerParams(use_tc_tiling_on_sc=False),
  )
  def k(y_hbm, src_hbm, dst_hbm, z_hbm, out_hbm, sb0, db0, sb1, db1, g0, g1,
        acc, isem0, isem1, gsem0, gsem1):
    c = lax.axis_index("c")
    s = lax.axis_index("s")
    wid = s * _NC + c
    pltpu.async_copy(src_hbm.at[wid, 0], sb0, isem0)
    pltpu.async_copy(dst_hbm.at[wid, 0], db0, isem0)
    pltpu.async_copy(src_hbm.at[wid, 1], sb1, isem1)
    pltpu.async_copy(dst_hbm.at[wid, 1], db1, isem1)
    zr = _ACC // _NS
    pltpu.sync_copy(z_hbm.at[pl.ds(s * zr, zr)], acc.at[pl.ds(s * zr, zr)])
    plsc.subcore_barrier()

    for p in range(_NPH):
      sb, db, isem = (sb0, db0, isem0) if p % 2 == 0 else (sb1, db1, isem1)
      pltpu.make_async_copy(src_hbm.at[wid, p], sb, isem).wait()
      pltpu.make_async_copy(dst_hbm.at[wid, p], db, isem).wait()
      pltpu.async_copy(y_hbm.at[sb.at[0]], g0, gsem0)
      pltpu.async_copy(y_hbm.at[sb.at[1]], g1, gsem1)

      def step(i, carry, sb=sb, db=db):
        for b, (g, gs) in enumerate(((g0, gsem0), (g1, gsem1))):
          j = i * 2 + b
          pltpu.make_async_copy(y_hbm.at[sb.at[j]], g, gs).wait()
          pltpu.sync_copy(g, acc.at[db.at[j]], add=True)

          @pl.when(j + 2 < _PCH)
          def _():
            pltpu.async_copy(y_hbm.at[sb.at[j + 2]], g, gs)
        return carry

      lax.fori_loop(0, _PCH // 2, step, 0)
      if p + 2 < _NPH:
        pltpu.async_copy(src_hbm.at[wid, p + 2], sb, isem)
        pltpu.async_copy(dst_hbm.at[wid, p + 2], db, isem)

    plsc.subcore_barrier()
    orow = _ACC // _NS
    pltpu.sync_copy(acc.at[pl.ds(s * orow, orow)],
                    out_hbm.at[pl.ds(c * _ACC + s * orow, orow)])

  return k(y, src_t, dst_t, zeros)


def _tc_first(x, w_rel, w_root):
  dout = w_rel.shape[1]

  def body(x_r, wr_r, wo_r, y_r, r_r):
    xv = x_r[...]
    y_r[...] = jnp.dot(xv, wr_r[...], preferred_element_type=jnp.float32)
    r_r[...] = jnp.dot(xv, wo_r[...], preferred_element_type=jnp.float32)

  return pl.pallas_call(
      body,
      out_shape=(jax.ShapeDtypeStruct((_N, dout), jnp.float32),
                 jax.ShapeDtypeStruct((_N, dout), jnp.float32)),
  )(x, w_rel, w_root)


def _tc_mid(P, r, b2d, w_rel, w_root):
  dout = w_rel.shape[1]

  def body(p_r, r_r, b_r, wr_r, wo_r, y_r, q_r):
    h = p_r[:_N, :] + p_r[_ACC:_ACC + _N, :] + b_r[...] + r_r[...]
    h = jnp.maximum(h, 0.0)
    y_r[...] = jnp.dot(h, wr_r[...], preferred_element_type=jnp.float32)
    q_r[...] = jnp.dot(h, wo_r[...], preferred_element_type=jnp.float32)

  return pl.pallas_call(
      body,
      out_shape=(jax.ShapeDtypeStruct((_N, dout), jnp.float32),
                 jax.ShapeDtypeStruct((_N, dout), jnp.float32)),
  )(P, r, b2d, w_rel, w_root)


def _tc_head(P, r, b2d, batch2d, fc_w, fc_b2d):
  def body(p_r, r_r, b_r, bat_r, w_r, c_r, o_r):
    h = p_r[:_N, :] + p_r[_ACC:_ACC + _N, :] + b_r[...] + r_r[...]
    gid = lax.broadcasted_iota(jnp.int32, (_G, 1), 0)
    onehot = (bat_r[...] == gid).astype(jnp.float32)
    sums = jnp.dot(onehot, h, preferred_element_type=jnp.float32)
    cnt = jnp.sum(onehot, axis=1, keepdims=True)
    pooled = sums / jnp.maximum(cnt, 1.0)
    o_r[...] = jnp.dot(pooled, w_r[...],
                       preferred_element_type=jnp.float32) + c_r[...]

  return pl.pallas_call(
      body,
      out_shape=jax.ShapeDtypeStruct((_G, 1), jnp.float32),
  )(P, r, b2d, batch2d, fc_w, fc_b2d)


def kernel(x, edge_index, batch, W1_rel, b1_rel, W1_root, W2_rel, b2_rel,
           W2_root, W3_rel, b3_rel, W3_root, W4_rel, b4_rel, W4_root,
           fc_W, fc_b):
  src = edge_index[0]
  dst = edge_index[1]
  pad = _EPAD - src.shape[0]
  pad_ar = jnp.arange(pad, dtype=jnp.int32)
  src_t = jnp.concatenate(
      [src, pad_ar % _N]).reshape(_NW, _NPH, _PCH, _CHUNK)
  dst_t = jnp.concatenate(
      [dst, _N + pad_ar % (_ACC - _N)]).reshape(_NW, _NPH, _PCH, _CHUNK)
  batch2d = batch.reshape(1, _N)

  y, r = _tc_first(x, W1_rel, W1_root)
  P = _sc_edge_aggregate(y, src_t, dst_t,
                         jnp.zeros((_ACC, W1_rel.shape[1]), jnp.float32))
  y, r = _tc_mid(P, r, b1_rel.reshape(1, -1), W2_rel, W2_root)
  P = _sc_edge_aggregate(y, src_t, dst_t,
                         jnp.zeros((_ACC, W2_rel.shape[1]), jnp.float32))
  y, r = _tc_mid(P, r, b2_rel.reshape(1, -1), W3_rel, W3_root)
  P = _sc_edge_aggregate(y, src_t, dst_t,
                         jnp.zeros((_ACC, W3_rel.shape[1]), jnp.float32))
  y, r = _tc_mid(P, r, b3_rel.reshape(1, -1), W4_rel, W4_root)
  P = _sc_edge_aggregate(y, src_t, dst_t,
                         jnp.zeros((_ACC, W4_rel.shape[1]), jnp.float32))
  return _tc_head(P, r, b4_rel.reshape(1, -1), batch2d, fc_W,
                  fc_b.reshape(1, 1))
